# baseline (speedup 1.0000x reference)
import functools

import jax
import jax.numpy as jnp
from jax import lax
from jax.experimental import pallas as pl
from jax.experimental.pallas import tpu as pltpu
from jax.experimental.pallas import tpu_sc as plsc

_VOCAB = 1000000
_D = 64
_SCALE = float(_D) ** 0.5

_NC = 2
_NS = 16
_NW = _NC * _NS

_BATCH = 16384
_SEQ = 50
_B = _BATCH * _SEQ
_C = 128
_NCHUNK_TOT = _B // _C
_PER_W = _NCHUNK_TOT // _NW
_NB = 4
_IT = _BATCH // _C
_ROWS_PER_J = (_D // 8) * _IT * 8


_NBLK = 3906


def _pack_body(tt_hbm, tail_hbm, out_hbm, a_v, b_v, tp_v, *sems):
    isems, osems = sems
    c = lax.axis_index("c")
    s = lax.axis_index("s")
    wid = s * _NC + c
    lane = lax.iota(jnp.int32, 16)
    xors = [jnp.bitwise_xor(lane, k) for k in range(16)]

    extra = jnp.where(wid < _NBLK - 122 * _NW, wid, _NBLK - 122 * _NW)
    start = wid * 122 + extra
    count = 122 + jnp.where(wid < _NBLK - 122 * _NW, 1, 0)

    def issue_in(g, r):
        pltpu.async_copy(tt_hbm.at[:, pl.ds(g * 256, 128)], a_v.at[r], isems.at[r])
        pltpu.async_copy(
            tt_hbm.at[:, pl.ds(g * 256 + 128, 128)], b_v.at[r], isems.at[r]
        )

    def wait_in(r):
        pltpu.make_async_copy(
            tt_hbm.at[:, pl.ds(0, 128)], a_v.at[r], isems.at[r]
        ).wait()
        pltpu.make_async_copy(
            tt_hbm.at[:, pl.ds(0, 128)], b_v.at[r], isems.at[r]
        ).wait()

    def transpose(r):
        @plsc.parallel_loop(0, 32, 1)
        def _(t):
            ob = lax.shift_right_logical(t, 2)
            cb = lax.bitwise_and(t, 3) * 16
            lo = ob * 16 + lane
            for k in range(16):
                cr = xors[k] + cb
                va = plsc.load_gather(a_v.at[r], [cr, lo])
                plsc.store_scatter(tp_v.at[r], [lo, cr], va)
                vb = plsc.load_gather(b_v.at[r], [cr, lo])
                plsc.store_scatter(tp_v.at[r], [lo, cr + _D], vb)

    def issue_out(g, r):
        pltpu.async_copy(tp_v.at[r], out_hbm.at[pl.ds(g * 128, 128)], osems.at[r])

    def wait_out(r):
        pltpu.make_async_copy(
            tp_v.at[r], out_hbm.at[pl.ds(0, 128)], osems.at[r]
        ).wait()

    issue_in(start, 0)

    @pl.when(count > 1)
    def _():
        issue_in(start + 1, 1)

    def blk(i, carry):
        r = lax.rem(i, 2)

        @pl.when(i >= 2)
        def _():
            wait_out(r)

        wait_in(r)
        transpose(r)
        issue_out(start + i, r)

        @pl.when(i + 2 < count)
        def _():
            issue_in(start + i + 2, r)

        return carry

    lax.fori_loop(0, count, blk, 0)
    wait_out(lax.rem(count - 1, 2))

    @pl.when(count > 1)
    def _():
        wait_out(lax.rem(count, 2))

    @pl.when(wid == 0)
    def _():
        pltpu.sync_copy(tail_hbm, out_hbm.at[pl.ds(_NBLK * 128, _D)])


@jax.jit
def _pack(tt, tail2):
    mesh = plsc.VectorSubcoreMesh(core_axis_name="c", subcore_axis_name="s")
    k = functools.partial(
        pl.kernel,
        out_type=jax.ShapeDtypeStruct((_NBLK * 128 + _D, 128), jnp.float32),
        mesh=mesh,
        scratch_types=[
            pltpu.VMEM((2, _D, 128), jnp.float32),
            pltpu.VMEM((2, _D, 128), jnp.float32),
            pltpu.VMEM((2, 128, 128), jnp.float32),
            pltpu.SemaphoreType.DMA((2,)),
            pltpu.SemaphoreType.DMA((2,)),
        ],
        compiler_params=pltpu.CompilerParams(
            use_tc_tiling_on_sc=True, needs_layout_passes=False
        ),
    )(_pack_body)
    return k(tt, tail2)


def _sc_body(table_hbm, idx_hbm, out_hbm, idx_v, u_v, rows_v, tb_v, *sems):
    gsems = sems[:_NB]
    osems = sems[_NB:]

    c = lax.axis_index("c")
    s = lax.axis_index("s")
    wid = s * _NC + c
    d0 = wid * _PER_W

    pltpu.sync_copy(idx_hbm.at[pl.ds(d0, _PER_W)], idx_v)

    lane = lax.iota(jnp.int32, 16)
    xors = [jnp.bitwise_xor(lane, k) for k in range(16)]

    def issue_gather(m, b):
        @plsc.parallel_loop(0, _C // 16, 1)
        def _(h):
            sl = pl.ds(h * 16, 16)
            raw = idx_v[m, sl]
            u_v[b, sl] = (
                lax.shift_left(lax.shift_right_logical(raw, 8), 8)
                + lax.shift_left(lax.bitwise_and(raw, 127), 1)
                + lax.bitwise_and(lax.shift_right_logical(raw, 7), 1)
            )

        pltpu.async_copy(table_hbm.at[u_v.at[b]], rows_v.at[b], gsems[b])

    def drain_gather(b):
        pltpu.make_async_copy(
            table_hbm.at[pl.ds(0, _C)], rows_v.at[b], gsems[b]
        ).wait()

    def transcale(m, b):
        @plsc.parallel_loop(0, (_C // 16) * (_D // 16), 1)
        def _(t):
            h = lax.shift_right_logical(t, 2)
            cb = lax.bitwise_and(t, (_D // 16) - 1) * 16
            li = lane + h * 16
            for k in range(16):
                crow = xors[k] + cb
                vals = plsc.load_gather(rows_v.at[b], [li, crow])
                plsc.store_scatter(tb_v.at[b], [crow, li], vals * _SCALE)

    def issue_out(m, b):
        d = d0 + m
        base = (
            lax.shift_right_logical(d, 7) * _ROWS_PER_J
            + lax.bitwise_and(d, _IT - 1) * 8
        )
        for tr in range(_D // 8):
            pltpu.async_copy(
                tb_v.at[b].at[pl.ds(tr * 8, 8)],
                out_hbm.at[pl.ds(base + tr * (_IT * 8), 8)],
                osems[b],
            )

    def drain_out(b):
        pltpu.make_async_copy(
            tb_v.at[b], out_hbm.at[pl.ds(0, _D)], osems[b]
        ).wait()

    for b in range(_NB):
        issue_gather(jnp.int32(b), b)

    for b in range(_NB):
        m = jnp.int32(b)
        drain_gather(b)
        transcale(m, b)
        issue_out(m, b)
        issue_gather(m + _NB, b)

    def outer(o, carry):
        for b in range(_NB):
            m = o * _NB + b
            drain_out(b)
            drain_gather(b)
            transcale(m, b)
            issue_out(m, b)
            issue_gather(m + _NB, b)
        return carry

    lax.fori_loop(1, _PER_W // _NB - 1, outer, 0)

    for b in range(_NB):
        m = jnp.int32(_PER_W - _NB + b)
        drain_out(b)
        drain_gather(b)
        transcale(m, b)
        issue_out(m, b)
    for b in range(_NB):
        drain_out(b)


@jax.jit
def _embed(table2, idx2d):
    mesh = plsc.VectorSubcoreMesh(core_axis_name="c", subcore_axis_name="s")
    k = functools.partial(
        pl.kernel,
        out_type=jax.ShapeDtypeStruct((_SEQ * _ROWS_PER_J, _C), jnp.float32),
        mesh=mesh,
        scratch_types=[
            pltpu.VMEM((_PER_W, _C), jnp.int32),
            pltpu.VMEM((_NB, _C), jnp.int32),
            pltpu.VMEM((_NB, _C, _D), jnp.float32),
            pltpu.VMEM((_NB, _D, _C), jnp.float32),
        ]
        + [pltpu.SemaphoreType.DMA] * (2 * _NB),
        compiler_params=pltpu.CompilerParams(
            use_tc_tiling_on_sc=False, needs_layout_passes=False
        ),
    )(_sc_body)
    return k(table2, idx2d)


def kernel(xb, table):
    tail2 = jnp.pad(
        lax.slice(table, (_NBLK * 256, 0), (_VOCAB, _D)), ((0, 0), (0, _D))
    )
    table2 = _pack(jnp.transpose(table), tail2)
    table2 = table2.reshape((_NBLK * 256 + 2 * _D, _D))
    idx2d = jnp.transpose(xb).astype(jnp.int32).reshape(_NCHUNK_TOT, _C)
    flat = _embed(table2, idx2d)
    a = flat.reshape(_SEQ, _D // 8, _IT, 8, _C)
    return a.transpose(2, 4, 0, 1, 3).reshape(_BATCH, _SEQ, _D)


# speedup vs baseline: 1.7418x; 1.7418x over previous
import functools

import jax
import jax.numpy as jnp
from jax import lax
from jax.experimental import pallas as pl
from jax.experimental.pallas import tpu as pltpu
from jax.experimental.pallas import tpu_sc as plsc

_VOCAB = 1000000
_D = 64
_SCALE = float(_D) ** 0.5

_NC = 2
_NS = 16
_NW = _NC * _NS

_BATCH = 16384
_SEQ = 50
_B = _BATCH * _SEQ
_C = 128
_NCHUNK_TOT = _B // _C
_PER_W = _NCHUNK_TOT // _NW
_NB = 4
_IT = _BATCH // _C
_ROWS_PER_J = (_D // 8) * _IT * 8


_NBLK = 3906


def _pack_body(tt_hbm, tail_hbm, out_hbm, a_v, b_v, tp_v, *sems):
    isems, osems = sems
    c = lax.axis_index("c")
    s = lax.axis_index("s")
    wid = s * _NC + c
    lane = lax.iota(jnp.int32, 16)
    xors = [jnp.bitwise_xor(lane, k) for k in range(16)]

    extra = jnp.where(wid < _NBLK - 122 * _NW, wid, _NBLK - 122 * _NW)
    start = wid * 122 + extra
    count = 122 + jnp.where(wid < _NBLK - 122 * _NW, 1, 0)

    def issue_in(g, r):
        pltpu.async_copy(tt_hbm.at[:, pl.ds(g * 256, 128)], a_v.at[r], isems.at[r])
        pltpu.async_copy(
            tt_hbm.at[:, pl.ds(g * 256 + 128, 128)], b_v.at[r], isems.at[r]
        )

    def wait_in(r):
        pltpu.make_async_copy(
            tt_hbm.at[:, pl.ds(0, 128)], a_v.at[r], isems.at[r]
        ).wait()
        pltpu.make_async_copy(
            tt_hbm.at[:, pl.ds(0, 128)], b_v.at[r], isems.at[r]
        ).wait()

    def transpose(r):
        @plsc.parallel_loop(0, 32, 1)
        def _(t):
            ob = lax.shift_right_logical(t, 2)
            cb = lax.bitwise_and(t, 3) * 16
            lo = ob * 16 + lane
            for k in range(16):
                cr = xors[k] + cb
                va = plsc.load_gather(a_v.at[r], [cr, lo])
                plsc.store_scatter(tp_v.at[r], [lo, cr], va)
                vb = plsc.load_gather(b_v.at[r], [cr, lo])
                plsc.store_scatter(tp_v.at[r], [lo, cr + _D], vb)

    def issue_out(g, r):
        pltpu.async_copy(tp_v.at[r], out_hbm.at[pl.ds(g * 128, 128)], osems.at[r])

    def wait_out(r):
        pltpu.make_async_copy(
            tp_v.at[r], out_hbm.at[pl.ds(0, 128)], osems.at[r]
        ).wait()

    issue_in(start, 0)

    @pl.when(count > 1)
    def _():
        issue_in(start + 1, 1)

    def blk(i, carry):
        r = lax.rem(i, 2)

        @pl.when(i >= 2)
        def _():
            wait_out(r)

        wait_in(r)
        transpose(r)
        issue_out(start + i, r)

        @pl.when(i + 2 < count)
        def _():
            issue_in(start + i + 2, r)

        return carry

    lax.fori_loop(0, count, blk, 0)
    wait_out(lax.rem(count - 1, 2))

    @pl.when(count > 1)
    def _():
        wait_out(lax.rem(count, 2))

    @pl.when(wid == 0)
    def _():
        pltpu.sync_copy(tail_hbm, out_hbm.at[pl.ds(_NBLK * 128, _D)])


@jax.jit
def _pack(tt, tail2):
    mesh = plsc.VectorSubcoreMesh(core_axis_name="c", subcore_axis_name="s")
    k = functools.partial(
        pl.kernel,
        out_type=jax.ShapeDtypeStruct((_NBLK * 128 + _D, 128), jnp.float32),
        mesh=mesh,
        scratch_types=[
            pltpu.VMEM((2, _D, 128), jnp.float32),
            pltpu.VMEM((2, _D, 128), jnp.float32),
            pltpu.VMEM((2, 128, 128), jnp.float32),
            pltpu.SemaphoreType.DMA((2,)),
            pltpu.SemaphoreType.DMA((2,)),
        ],
        compiler_params=pltpu.CompilerParams(
            use_tc_tiling_on_sc=True, needs_layout_passes=False
        ),
    )(_pack_body)
    return k(tt, tail2)


def _sc_body(table_hbm, idx_hbm, out_hbm, idx_v, u_v, rows_v, tb_v, *sems):
    gsems = sems[:_NB]
    osems = sems[_NB:]

    c = lax.axis_index("c")
    s = lax.axis_index("s")
    wid = s * _NC + c
    d0 = wid * _PER_W

    pltpu.sync_copy(idx_hbm.at[pl.ds(d0, _PER_W)], idx_v)

    lane = lax.iota(jnp.int32, 16)
    xors = [jnp.bitwise_xor(lane, k) for k in range(16)]

    def issue_gather(m, b):
        @plsc.parallel_loop(0, _C // 16, 1)
        def _(h):
            sl = pl.ds(h * 16, 16)
            raw = idx_v[m, sl]
            u_v[b, sl] = (
                lax.shift_left(lax.shift_right_logical(raw, 8), 7)
                + lax.bitwise_and(raw, 127)
            )

        pltpu.async_copy(table_hbm.at[u_v.at[b]], rows_v.at[b], gsems[b])

    def drain_gather(b):
        pltpu.make_async_copy(
            table_hbm.at[pl.ds(0, _C)], rows_v.at[b], gsems[b]
        ).wait()

    def transcale(m, b):
        @plsc.parallel_loop(0, (_C // 16) * (_D // 16), 1)
        def _(t):
            h = lax.shift_right_logical(t, 2)
            cb = lax.bitwise_and(t, (_D // 16) - 1) * 16
            raw = idx_v[m, pl.ds(h * 16, 16)]
            par = lax.shift_left(
                lax.bitwise_and(lax.shift_right_logical(raw, 7), 1), 6
            )
            li = lane + h * 16
            for k in range(16):
                crow = xors[k] + cb
                vals = plsc.load_gather(rows_v.at[b], [li, crow + par])
                plsc.store_scatter(tb_v.at[b], [crow, li], vals * _SCALE)

    def issue_out(m, b):
        d = d0 + m
        base = (
            lax.shift_right_logical(d, 7) * _ROWS_PER_J
            + lax.bitwise_and(d, _IT - 1) * 8
        )
        for tr in range(_D // 8):
            pltpu.async_copy(
                tb_v.at[b].at[pl.ds(tr * 8, 8)],
                out_hbm.at[pl.ds(base + tr * (_IT * 8), 8)],
                osems[b],
            )

    def drain_out(b):
        pltpu.make_async_copy(
            tb_v.at[b], out_hbm.at[pl.ds(0, _D)], osems[b]
        ).wait()

    for b in range(_NB):
        issue_gather(jnp.int32(b), b)

    for b in range(_NB):
        m = jnp.int32(b)
        drain_gather(b)
        transcale(m, b)
        issue_out(m, b)
        issue_gather(m + _NB, b)

    def outer(o, carry):
        for b in range(_NB):
            m = o * _NB + b
            drain_out(b)
            drain_gather(b)
            transcale(m, b)
            issue_out(m, b)
            issue_gather(m + _NB, b)
        return carry

    lax.fori_loop(1, _PER_W // _NB - 1, outer, 0)

    for b in range(_NB):
        m = jnp.int32(_PER_W - _NB + b)
        drain_out(b)
        drain_gather(b)
        transcale(m, b)
        issue_out(m, b)
    for b in range(_NB):
        drain_out(b)


@jax.jit
def _embed(table2, idx2d):
    mesh = plsc.VectorSubcoreMesh(core_axis_name="c", subcore_axis_name="s")
    k = functools.partial(
        pl.kernel,
        out_type=jax.ShapeDtypeStruct((_SEQ * _ROWS_PER_J, _C), jnp.float32),
        mesh=mesh,
        scratch_types=[
            pltpu.VMEM((_PER_W, _C), jnp.int32),
            pltpu.VMEM((_NB, _C), jnp.int32),
            pltpu.VMEM((_NB, _C, _C), jnp.float32),
            pltpu.VMEM((_NB, _D, _C), jnp.float32),
        ]
        + [pltpu.SemaphoreType.DMA] * (2 * _NB),
        compiler_params=pltpu.CompilerParams(
            use_tc_tiling_on_sc=False, needs_layout_passes=False
        ),
    )(_sc_body)
    return k(table2, idx2d)


def kernel(xb, table):
    tail2 = jnp.pad(
        lax.slice(table, (_NBLK * 256, 0), (_VOCAB, _D)), ((0, 0), (0, _D))
    )
    table2 = _pack(jnp.transpose(table), tail2)
    idx2d = jnp.transpose(xb).astype(jnp.int32).reshape(_NCHUNK_TOT, _C)
    flat = _embed(table2, idx2d)
    a = flat.reshape(_SEQ, _D // 8, _IT, 8, _C)
    return a.transpose(2, 4, 0, 1, 3).reshape(_BATCH, _SEQ, _D)
